# baseline (speedup 1.0000x reference)
import functools

import jax
import jax.numpy as jnp
from jax import lax
from jax.experimental import pallas as pl
from jax.experimental.pallas import tpu as pltpu
from jax.experimental.pallas import tpu_sc as plsc

B = 16384
C = 128
CAM = 1000
VIEW = 100
NC = 2
NS = 16
NW = NC * NS
BPW = B // NW
CH = 256
NCHUNK = BPW // CH
CROWS = 64


def _sie_body(x_hbm, cam_hbm, view_hbm, camtab_hbm, viewtab_hbm, out_hbm,
              cam_idx_v, view_idx_v, ctab_sh, vtab_sh, xbuf,
              sem_i, sem_t, sems_x, sems_c, sems_v, sems_o):
    s = lax.axis_index("s")
    wid = s * NC + lax.axis_index("c")

    @pl.when(s < NS - 1)
    def _stage_cam_main():
        pltpu.sync_copy(camtab_hbm.at[pl.ds(s * CROWS, CROWS)],
                        ctab_sh.at[pl.ds(s * CROWS, CROWS)])

    @pl.when(s == NS - 1)
    def _stage_cam_tail():
        pltpu.sync_copy(camtab_hbm.at[pl.ds(CROWS * (NS - 1), CAM - CROWS * (NS - 1))],
                        ctab_sh.at[pl.ds(CROWS * (NS - 1), CAM - CROWS * (NS - 1))])

    @pl.when(s == 0)
    def _stage_view_table():
        pltpu.sync_copy(viewtab_hbm, vtab_sh)

    ci = pltpu.async_copy(cam_hbm.at[wid], cam_idx_v, sem_i)
    vi = pltpu.async_copy(view_hbm.at[wid], view_idx_v, sem_i)
    xc = [pltpu.async_copy(x_hbm.at[wid].at[pl.ds(i * CH, CH)],
                           xbuf.at[pl.ds(i * CH, CH)], sems_x[i])
          for i in range(NCHUNK)]
    plsc.subcore_barrier()
    ci.wait()
    vi.wait()

    gathers = []
    for i in range(NCHUNK):
        xc[i].wait()
        sl = pl.ds(i * CH, CH)
        cc = pltpu.async_copy(ctab_sh.at[cam_idx_v.at[sl]], xbuf.at[sl],
                              sems_c[i], add=True)
        cv = pltpu.async_copy(vtab_sh.at[view_idx_v.at[sl]], xbuf.at[sl],
                              sems_v[i], add=True)
        gathers.append((cc, cv))
    wbs = []
    for i in range(NCHUNK):
        cc, cv = gathers[i]
        cc.wait()
        cv.wait()
        sl = pl.ds(i * CH, CH)
        wbs.append(pltpu.async_copy(xbuf.at[sl], out_hbm.at[wid].at[sl],
                                    sems_o[i]))
    for w in wbs:
        w.wait()


@functools.partial(jax.jit, static_argnames=())
def _sie(x, cam_label, view_label, camera_embedding, view_embedding):
    run = pl.kernel(
        _sie_body,
        out_type=jax.ShapeDtypeStruct((NW, BPW, C), jnp.float32),
        mesh=plsc.VectorSubcoreMesh(core_axis_name="c", subcore_axis_name="s"),
        scratch_types=[
            pltpu.VMEM((BPW,), jnp.int32),
            pltpu.VMEM((BPW,), jnp.int32),
            pltpu.VMEM_SHARED((CAM, C), jnp.float32),
            pltpu.VMEM_SHARED((VIEW, C), jnp.float32),
            pltpu.VMEM((BPW, C), jnp.float32),
            pltpu.SemaphoreType.DMA,
            pltpu.SemaphoreType.DMA,
            [pltpu.SemaphoreType.DMA] * NCHUNK,
            [pltpu.SemaphoreType.DMA] * NCHUNK,
            [pltpu.SemaphoreType.DMA] * NCHUNK,
            [pltpu.SemaphoreType.DMA] * NCHUNK,
        ],
    )
    out = run(x.reshape(NW, BPW, C),
              cam_label.reshape(NW, BPW),
              view_label.reshape(NW, BPW),
              camera_embedding, view_embedding)
    return out.reshape(B, C)


def kernel(x, cam_label, view_label, camera_embedding, view_embedding):
    return _sie(x, cam_label.astype(jnp.int32), view_label.astype(jnp.int32),
                camera_embedding, view_embedding)


# speedup vs baseline: 1.0738x; 1.0738x over previous
import functools

import jax
import jax.numpy as jnp
from jax import lax
from jax.experimental import pallas as pl
from jax.experimental.pallas import tpu as pltpu
from jax.experimental.pallas import tpu_sc as plsc

B = 16384
C = 128
VIEW = 100
NC = 2
NS = 16
NW = NC * NS
BPW = B // NW


def _sie_body(x_hbm, cam_hbm, view_hbm, camtab_hbm, viewtab_hbm, out_hbm,
              cam_idx_v, view_idx_v, vtab_sh, xbuf,
              sem_i, sem_t, sem_x, sem_c, sem_v):
    wid = lax.axis_index("s") * NC + lax.axis_index("c")

    ci = pltpu.async_copy(cam_hbm.at[wid], cam_idx_v, sem_i)
    vi = pltpu.async_copy(view_hbm.at[wid], view_idx_v, sem_i)
    cx = pltpu.async_copy(x_hbm.at[wid], xbuf, sem_x)

    s = lax.axis_index("s")

    @pl.when(s < 3)
    def _stage_view_table():
        q = pl.multiple_of(s * 32, 32)
        pltpu.sync_copy(viewtab_hbm.at[pl.ds(q, 32)],
                        vtab_sh.at[pl.ds(q, 32)])

    @pl.when(s == 3)
    def _stage_view_tail():
        pltpu.sync_copy(viewtab_hbm.at[pl.ds(96, VIEW - 96)],
                        vtab_sh.at[pl.ds(96, VIEW - 96)])

    ci.wait()
    vi.wait()
    cx.wait()
    cc = pltpu.async_copy(camtab_hbm.at[cam_idx_v], xbuf, sem_c, add=True)
    plsc.subcore_barrier()
    cv = pltpu.async_copy(vtab_sh.at[view_idx_v], xbuf, sem_v, add=True)
    cc.wait()
    cv.wait()
    pltpu.sync_copy(xbuf, out_hbm.at[wid])


@functools.partial(jax.jit, static_argnames=())
def _sie(x, cam_label, view_label, camera_embedding, view_embedding):
    run = pl.kernel(
        _sie_body,
        out_type=jax.ShapeDtypeStruct((NW, BPW, C), jnp.float32),
        mesh=plsc.VectorSubcoreMesh(core_axis_name="c", subcore_axis_name="s"),
        scratch_types=[
            pltpu.VMEM((BPW,), jnp.int32),
            pltpu.VMEM((BPW,), jnp.int32),
            pltpu.VMEM_SHARED((VIEW, C), jnp.float32),
            pltpu.VMEM((BPW, C), jnp.float32),
            pltpu.SemaphoreType.DMA,
            pltpu.SemaphoreType.DMA,
            pltpu.SemaphoreType.DMA,
            pltpu.SemaphoreType.DMA,
            pltpu.SemaphoreType.DMA,
        ],
    )
    out = run(x.reshape(NW, BPW, C),
              cam_label.reshape(NW, BPW),
              view_label.reshape(NW, BPW),
              camera_embedding, view_embedding)
    return out.reshape(B, C)


def kernel(x, cam_label, view_label, camera_embedding, view_embedding):
    return _sie(x, cam_label.astype(jnp.int32), view_label.astype(jnp.int32),
                camera_embedding, view_embedding)
